# baseline (speedup 1.0000x reference)
import functools

import jax
import jax.numpy as jnp
from jax import lax
from jax.experimental import pallas as pl
from jax.experimental.pallas import tpu as pltpu
from jax.experimental.pallas import tpu_sc as plsc

H = 128
EPS = 1e-12
GATHER_W = 128
N_CHUNKS = 4


def _pack_table(word_table):
    u = lax.bitcast_convert_type(word_table, jnp.int32)
    b = lax.shift_right_logical(
        u + 0x7FFF + (lax.shift_right_logical(u, 16) & 1), 16)
    return b[:, : H // 2] | lax.shift_left(b[:, H // 2:], 16)


def _sc_gather(table_pk, ids_2d, n):
    hw = H // 2

    @functools.partial(
        pl.kernel,
        out_type=jax.ShapeDtypeStruct((n // 2, H), jnp.int32),
        mesh=plsc.VectorSubcoreMesh(core_axis_name="core",
                                    subcore_axis_name="subcore"),
        compiler_params=pltpu.CompilerParams(use_tc_tiling_on_sc=False),
    )
    def k(table_hbm, i_hbm, o_hbm):
        def body(i_vmem, o_vmem):
            pltpu.sync_copy(table_hbm.at[i_vmem.at[0]], o_vmem)

        pltpu.emit_pipeline(
            body,
            grid=(n // GATHER_W,),
            in_specs=[pl.BlockSpec((1, GATHER_W), index_map=lambda i: (0, i))],
            out_specs=[pl.BlockSpec((GATHER_W, hw),
                                    index_map=lambda i: (i // 2, i % 2))],
            core_axis_name=("core", "subcore"),
            dimension_semantics=(pltpu.PARALLEL,),
        )(i_hbm, o_hbm)

    return k(table_pk, ids_2d)


def _ln_body(g_ref, pos_ref, tok_ref, gamma_ref, beta_ref, o_ref):
    gi = g_ref[...]
    hs = gi.shape[0] // 2
    q = jnp.concatenate([gi[:hs, : H // 2], gi[:hs, H // 2:],
                         gi[hs:, : H // 2], gi[hs:, H // 2:]], axis=0)
    lo = lax.bitcast_convert_type(lax.shift_left(q, 16), jnp.float32)
    hi = lax.bitcast_convert_type(
        jnp.bitwise_and(q, jnp.int32(-65536)), jnp.float32)
    x = (jnp.concatenate([lo, hi], axis=-1)
         + pos_ref[...] + tok_ref[...][0][None, :])
    ones = jnp.ones((H, H), dtype=jnp.bfloat16)
    xb = x.astype(jnp.bfloat16)
    s1 = jax.lax.dot(xb, ones, preferred_element_type=jnp.float32)
    s2 = jax.lax.dot(xb * xb, ones, preferred_element_type=jnp.float32)
    mean = s1 * (1.0 / H)
    var = s2 * (1.0 / H) - mean * mean
    inv = lax.rsqrt(var + EPS)
    a = inv * gamma_ref[...][0][None, :]
    b = beta_ref[...][0][None, :] - mean * a
    o_ref[...] = (x * a + b)[None]


def _ln_body_acc(g_ref, pos_ref, tok_ref, gamma_ref, beta_ref, acc_ref,
                 o_ref):
    del acc_ref
    _ln_body(g_ref, pos_ref, tok_ref, gamma_ref, beta_ref, o_ref)


def _tc_ln_chunk(g, pos, tok, gamma2d, beta2d, out_b, seq, chunk, buf):
    rows = g.shape[0]
    rows_blk = seq // 2
    bc = rows // rows_blk
    base = chunk * bc
    in_specs = [
        pl.BlockSpec((rows_blk, H), lambda i: (i, 0)),
        pl.BlockSpec((seq, H), lambda i: (0, 0)),
        pl.BlockSpec((2, H), lambda i: (0, 0)),
        pl.BlockSpec((1, H), lambda i: (0, 0)),
        pl.BlockSpec((1, H), lambda i: (0, 0)),
    ]
    args = [g, pos, tok, gamma2d, beta2d]
    body = _ln_body
    kwargs = {}
    if buf is not None:
        in_specs.append(pl.BlockSpec(memory_space=pl.ANY))
        args.append(buf)
        body = _ln_body_acc
        kwargs = dict(input_output_aliases={5: 0})
    return pl.pallas_call(
        body,
        grid=(bc,),
        in_specs=in_specs,
        out_specs=pl.BlockSpec((1, seq, H), lambda i: (base + i, 0, 0)),
        out_shape=jax.ShapeDtypeStruct((out_b, seq, H), jnp.float32),
        **kwargs,
    )(*args)


def kernel(input_ids, word_table, pos_table, tok_table, gamma, beta):
    B, S = input_ids.shape
    ids = input_ids.astype(jnp.int32)
    gamma2d, beta2d = gamma.reshape(1, H), beta.reshape(1, H)
    wt_pk = _pack_table(word_table)
    bc = B // N_CHUNKS
    gs = []
    for k in range(N_CHUNKS):
        ids_k = ids[k * bc:(k + 1) * bc].reshape(1, bc * S)
        gs.append(_sc_gather(wt_pk, ids_k, bc * S))
    out = None
    for k in range(N_CHUNKS):
        out = _tc_ln_chunk(gs[k], pos_table, tok_table, gamma2d, beta2d,
                           B, S, k, out)
    return out


# speedup vs baseline: 2.1347x; 2.1347x over previous
import functools

import jax
import jax.numpy as jnp
from jax import lax
from jax.experimental import pallas as pl
from jax.experimental.pallas import tpu as pltpu
from jax.experimental.pallas import tpu_sc as plsc

H = 128
EPS = 1e-12
GATHER_W = 128
N_CHUNKS = 4


def _pack_table(word_table):
    u = lax.bitcast_convert_type(word_table, jnp.int32)
    b = lax.shift_right_logical(
        u + 0x7FFF + (lax.shift_right_logical(u, 16) & 1), 16)
    return b[:, : H // 2] | lax.shift_left(b[:, H // 2:], 16)


def _sc_gather(table_pk, ids_2d, n):
    hw = H // 2

    @functools.partial(
        pl.kernel,
        out_type=jax.ShapeDtypeStruct((n // 2, H), jnp.int32),
        mesh=plsc.VectorSubcoreMesh(core_axis_name="core",
                                    subcore_axis_name="subcore"),
        compiler_params=pltpu.CompilerParams(use_tc_tiling_on_sc=False),
    )
    def k(table_hbm, i_hbm, o_hbm):
        def body(i_vmem, o_vmem):
            pltpu.sync_copy(table_hbm.at[i_vmem.at[0]], o_vmem)

        pltpu.emit_pipeline(
            body,
            grid=(n // GATHER_W,),
            in_specs=[pl.BlockSpec((1, GATHER_W), index_map=lambda i: (0, i))],
            out_specs=[pl.BlockSpec((GATHER_W, hw),
                                    index_map=lambda i: (i // 2, i % 2))],
            core_axis_name=("core", "subcore"),
            dimension_semantics=(pltpu.PARALLEL,),
        )(i_hbm, o_hbm)

    return k(table_pk, ids_2d)


def _ln_body(g_ref, pos_ref, tok_ref, gamma_ref, beta_ref, o_ref):
    gi = g_ref[...]
    nseq = o_ref.shape[0]
    seq = o_ref.shape[1]
    hs = seq // 4
    lo = lax.bitcast_convert_type(lax.shift_left(gi, 16), jnp.float32)
    hi = lax.bitcast_convert_type(
        jnp.bitwise_and(gi, jnp.int32(-65536)), jnp.float32)
    parts = []
    for j in range(nseq):
        for g2 in range(2):
            r0 = j * (seq // 2) + g2 * hs
            for h in range(2):
                c = slice(h * (H // 2), (h + 1) * (H // 2))
                parts.append(jnp.concatenate(
                    [lo[r0:r0 + hs, c], hi[r0:r0 + hs, c]], axis=-1))
    x = jnp.concatenate(parts, axis=0)
    pt = pos_ref[...] + tok_ref[...][0][None, :]
    x = (x.reshape(nseq, seq, H) + pt[None]).reshape(nseq * seq, H)
    ones = jnp.ones((H, H), dtype=jnp.bfloat16)
    xb = x.astype(jnp.bfloat16)
    s1 = jax.lax.dot(xb, ones, preferred_element_type=jnp.float32)
    s2 = jax.lax.dot(xb * xb, ones, preferred_element_type=jnp.float32)
    mean = s1 * (1.0 / H)
    var = s2 * (1.0 / H) - mean * mean
    inv = lax.rsqrt(var + EPS)
    y = (x - mean) * inv
    o_ref[...] = (y * gamma_ref[...][0][None, :]
                  + beta_ref[...][0][None, :]).reshape(nseq, seq, H)


def _ln_body_acc(g_ref, pos_ref, tok_ref, gamma_ref, beta_ref, acc_ref,
                 o_ref):
    del acc_ref
    _ln_body(g_ref, pos_ref, tok_ref, gamma_ref, beta_ref, o_ref)


def _tc_ln_chunk(g, pos, tok, gamma2d, beta2d, out_b, seq, chunk, buf):
    nseq = 4
    rows = g.shape[0]
    rows_blk = nseq * seq // 2
    bc = rows // rows_blk
    base = chunk * bc
    in_specs = [
        pl.BlockSpec((rows_blk, H), lambda i: (i, 0)),
        pl.BlockSpec((seq, H), lambda i: (0, 0)),
        pl.BlockSpec((2, H), lambda i: (0, 0)),
        pl.BlockSpec((1, H), lambda i: (0, 0)),
        pl.BlockSpec((1, H), lambda i: (0, 0)),
    ]
    args = [g, pos, tok, gamma2d, beta2d]
    body = _ln_body
    kwargs = {}
    if buf is not None:
        in_specs.append(pl.BlockSpec(memory_space=pl.ANY))
        args.append(buf)
        body = _ln_body_acc
        kwargs = dict(input_output_aliases={5: 0})
    return pl.pallas_call(
        body,
        grid=(bc,),
        in_specs=in_specs,
        out_specs=pl.BlockSpec((nseq, seq, H), lambda i: (base + i, 0, 0)),
        out_shape=jax.ShapeDtypeStruct((out_b, seq, H), jnp.float32),
        **kwargs,
    )(*args)


def kernel(input_ids, word_table, pos_table, tok_table, gamma, beta):
    B, S = input_ids.shape
    ids = input_ids.astype(jnp.int32)
    gamma2d, beta2d = gamma.reshape(1, H), beta.reshape(1, H)
    wt_pk = _pack_table(word_table)
    bc = B // N_CHUNKS
    gs = []
    for k in range(N_CHUNKS):
        ids_k = ids[k * bc:(k + 1) * bc].reshape(1, bc * S)
        gs.append(_sc_gather(wt_pk, ids_k, bc * S))
    out = None
    for k in range(N_CHUNKS):
        out = _tc_ln_chunk(gs[k], pos_table, tok_table, gamma2d, beta2d,
                           B, S, k, out)
    return out


# speedup vs baseline: 2.1678x; 1.0155x over previous
import functools

import jax
import jax.numpy as jnp
from jax import lax
from jax.experimental import pallas as pl
from jax.experimental.pallas import tpu as pltpu
from jax.experimental.pallas import tpu_sc as plsc

H = 128
EPS = 1e-12
GATHER_W = 128
N_CHUNKS = 4


def _pack_table(word_table):
    u = lax.bitcast_convert_type(word_table, jnp.int32)
    b = lax.shift_right_logical(
        u + 0x7FFF + (lax.shift_right_logical(u, 16) & 1), 16)
    return b[:, : H // 2] | lax.shift_left(b[:, H // 2:], 16)


def _sc_gather(table_pk, ids_2d, n):
    hw = H // 2

    @functools.partial(
        pl.kernel,
        out_type=jax.ShapeDtypeStruct((n // 2, H), jnp.int32),
        mesh=plsc.VectorSubcoreMesh(core_axis_name="core",
                                    subcore_axis_name="subcore"),
        compiler_params=pltpu.CompilerParams(use_tc_tiling_on_sc=False),
    )
    def k(table_hbm, i_hbm, o_hbm):
        def body(i_vmem, o_vmem):
            pltpu.sync_copy(table_hbm.at[i_vmem.at[0]], o_vmem)

        pltpu.emit_pipeline(
            body,
            grid=(n // GATHER_W,),
            in_specs=[pl.BlockSpec((1, GATHER_W), index_map=lambda i: (0, i))],
            out_specs=[pl.BlockSpec((GATHER_W, hw),
                                    index_map=lambda i: (i // 2, i % 2))],
            core_axis_name=("core", "subcore"),
            dimension_semantics=(pltpu.PARALLEL,),
        )(i_hbm, o_hbm)

    return k(table_pk, ids_2d)


def _ln_body(g_ref, pos_ref, tok_ref, gamma_ref, beta_ref, o_ref):
    gi = g_ref[...]
    nseq = o_ref.shape[0]
    seq = o_ref.shape[1]
    hs = seq // 4
    lo = lax.bitcast_convert_type(lax.shift_left(gi, 16), jnp.float32)
    hi = lax.bitcast_convert_type(
        jnp.bitwise_and(gi, jnp.int32(-65536)), jnp.float32)
    parts = []
    for j in range(nseq):
        for g2 in range(2):
            r0 = j * (seq // 2) + g2 * hs
            for h in range(2):
                c = slice(h * (H // 2), (h + 1) * (H // 2))
                parts.append(jnp.concatenate(
                    [lo[r0:r0 + hs, c], hi[r0:r0 + hs, c]], axis=-1))
    x = jnp.concatenate(parts, axis=0)
    pt = pos_ref[...] + tok_ref[...][0][None, :]
    x = (x.reshape(nseq, seq, H) + pt[None]).reshape(nseq * seq, H)
    onesh = jnp.full((H, H), 1.0 / H, dtype=jnp.bfloat16)
    xb = x.astype(jnp.bfloat16)
    mean = jax.lax.dot(xb, onesh, preferred_element_type=jnp.float32)
    ex2 = jax.lax.dot(xb * xb, onesh, preferred_element_type=jnp.float32)
    inv = lax.rsqrt(ex2 - mean * mean + EPS)
    y = (x - mean) * inv
    o_ref[...] = (y * gamma_ref[...][0][None, :]
                  + beta_ref[...][0][None, :]).reshape(nseq, seq, H)


def _ln_body_acc(g_ref, pos_ref, tok_ref, gamma_ref, beta_ref, acc_ref,
                 o_ref):
    del acc_ref
    _ln_body(g_ref, pos_ref, tok_ref, gamma_ref, beta_ref, o_ref)


def _tc_ln_chunk(g, pos, tok, gamma2d, beta2d, out_b, seq, chunk, buf):
    nseq = 4
    rows = g.shape[0]
    rows_blk = nseq * seq // 2
    bc = rows // rows_blk
    base = chunk * bc
    in_specs = [
        pl.BlockSpec((rows_blk, H), lambda i: (i, 0)),
        pl.BlockSpec((seq, H), lambda i: (0, 0)),
        pl.BlockSpec((2, H), lambda i: (0, 0)),
        pl.BlockSpec((1, H), lambda i: (0, 0)),
        pl.BlockSpec((1, H), lambda i: (0, 0)),
    ]
    args = [g, pos, tok, gamma2d, beta2d]
    body = _ln_body
    kwargs = {}
    if buf is not None:
        in_specs.append(pl.BlockSpec(memory_space=pl.ANY))
        args.append(buf)
        body = _ln_body_acc
        kwargs = dict(input_output_aliases={5: 0})
    return pl.pallas_call(
        body,
        grid=(bc,),
        in_specs=in_specs,
        out_specs=pl.BlockSpec((nseq, seq, H), lambda i: (base + i, 0, 0)),
        out_shape=jax.ShapeDtypeStruct((out_b, seq, H), jnp.float32),
        **kwargs,
    )(*args)


def kernel(input_ids, word_table, pos_table, tok_table, gamma, beta):
    B, S = input_ids.shape
    ids = input_ids.astype(jnp.int32)
    gamma2d, beta2d = gamma.reshape(1, H), beta.reshape(1, H)
    wt_pk = _pack_table(word_table)
    bc = B // N_CHUNKS
    gs = []
    for k in range(N_CHUNKS):
        ids_k = ids[k * bc:(k + 1) * bc].reshape(1, bc * S)
        gs.append(_sc_gather(wt_pk, ids_k, bc * S))
    out = None
    for k in range(N_CHUNKS):
        out = _tc_ln_chunk(gs[k], pos_table, tok_table, gamma2d, beta2d,
                           B, S, k, out)
    return out


# speedup vs baseline: 2.5564x; 1.1792x over previous
import functools

import jax
import jax.numpy as jnp
from jax import lax
from jax.experimental import pallas as pl
from jax.experimental.pallas import tpu as pltpu
from jax.experimental.pallas import tpu_sc as plsc

H = 128
EPS = 1e-12
GATHER_W = 128
N_CHUNKS = 4


def _pack_table(word_table):
    u = lax.bitcast_convert_type(word_table, jnp.int32)
    b = lax.shift_right_logical(
        u + 0x7FFF + (lax.shift_right_logical(u, 16) & 1), 16)
    return b[:, : H // 2] | lax.shift_left(b[:, H // 2:], 16)


def _sc_gather(table_pk, ids_2d, n):
    hw = H // 2

    @functools.partial(
        pl.kernel,
        out_type=jax.ShapeDtypeStruct((n // 2, H), jnp.int32),
        mesh=plsc.VectorSubcoreMesh(core_axis_name="core",
                                    subcore_axis_name="subcore"),
        compiler_params=pltpu.CompilerParams(use_tc_tiling_on_sc=False),
    )
    def k(table_hbm, i_hbm, o_hbm):
        def body(i_vmem, o_vmem):
            pltpu.sync_copy(table_hbm.at[i_vmem.at[0]], o_vmem)

        pltpu.emit_pipeline(
            body,
            grid=(n // GATHER_W,),
            in_specs=[pl.BlockSpec((1, GATHER_W), index_map=lambda i: (0, i))],
            out_specs=[pl.BlockSpec((GATHER_W, hw),
                                    index_map=lambda i: (i // 2, i % 2))],
            core_axis_name=("core", "subcore"),
            dimension_semantics=(pltpu.PARALLEL,),
        )(i_hbm, o_hbm)

    return k(table_pk, ids_2d)


def _ln_body(g_ref, pos_ref, tok_ref, gamma_ref, beta_ref, o_ref):
    gi = g_ref[...]
    nseq = o_ref.shape[0]
    seq = o_ref.shape[1]
    hs = seq // 4
    lo = lax.bitcast_convert_type(lax.shift_left(gi, 16), jnp.float32)
    hi = lax.bitcast_convert_type(
        jnp.bitwise_and(gi, jnp.int32(-65536)), jnp.float32)
    parts = []
    for j in range(nseq):
        for g2 in range(2):
            r0 = j * (seq // 2) + g2 * hs
            for h in range(2):
                c = slice(h * (H // 2), (h + 1) * (H // 2))
                parts.append(jnp.concatenate(
                    [lo[r0:r0 + hs, c], hi[r0:r0 + hs, c]], axis=-1))
    x = jnp.concatenate(parts, axis=0)
    pt = pos_ref[...] + tok_ref[...][0][None, :]
    x = (x.reshape(nseq, seq, H) + pt[None]).reshape(nseq * seq, H)
    onesh = jnp.full((H, H), 1.0 / H, dtype=jnp.bfloat16)
    xb = x.astype(jnp.bfloat16)
    mean = jax.lax.dot(xb, onesh, preferred_element_type=jnp.float32)
    ex2 = jax.lax.dot(xb * xb, onesh, preferred_element_type=jnp.float32)
    inv = lax.rsqrt(ex2 - mean * mean + EPS)
    y = (x - mean) * inv
    o_ref[...] = (y * gamma_ref[...][0][None, :]
                  + beta_ref[...][0][None, :]).reshape(nseq, seq, H)


def _ln_body_acc(g_ref, pos_ref, tok_ref, gamma_ref, beta_ref, acc_ref,
                 o_ref):
    del acc_ref
    _ln_body(g_ref, pos_ref, tok_ref, gamma_ref, beta_ref, o_ref)


def _tc_ln_chunk(g, pos, tok, gamma2d, beta2d, out_b, seq, chunk, buf):
    nseq = 8
    rows = g.shape[0]
    rows_blk = nseq * seq // 2
    bc = rows // rows_blk
    base = chunk * bc
    in_specs = [
        pl.BlockSpec((rows_blk, H), lambda i: (i, 0)),
        pl.BlockSpec((seq, H), lambda i: (0, 0)),
        pl.BlockSpec((2, H), lambda i: (0, 0)),
        pl.BlockSpec((1, H), lambda i: (0, 0)),
        pl.BlockSpec((1, H), lambda i: (0, 0)),
    ]
    args = [g, pos, tok, gamma2d, beta2d]
    body = _ln_body
    kwargs = {}
    if buf is not None:
        in_specs.append(pl.BlockSpec(memory_space=pl.ANY))
        args.append(buf)
        body = _ln_body_acc
        kwargs = dict(input_output_aliases={5: 0})
    return pl.pallas_call(
        body,
        grid=(bc,),
        in_specs=in_specs,
        out_specs=pl.BlockSpec((nseq, seq, H), lambda i: (base + i, 0, 0)),
        out_shape=jax.ShapeDtypeStruct((out_b, seq, H), jnp.float32),
        **kwargs,
    )(*args)


def kernel(input_ids, word_table, pos_table, tok_table, gamma, beta):
    B, S = input_ids.shape
    ids = input_ids.astype(jnp.int32)
    gamma2d, beta2d = gamma.reshape(1, H), beta.reshape(1, H)
    wt_pk = _pack_table(word_table)
    bc = B // N_CHUNKS
    gs = []
    for k in range(N_CHUNKS):
        ids_k = ids[k * bc:(k + 1) * bc].reshape(1, bc * S)
        gs.append(_sc_gather(wt_pk, ids_k, bc * S))
    out = None
    for k in range(N_CHUNKS):
        out = _tc_ln_chunk(gs[k], pos_table, tok_table, gamma2d, beta2d,
                           B, S, k, out)
    return out


# speedup vs baseline: 2.6176x; 1.0239x over previous
import functools

import jax
import jax.numpy as jnp
from jax import lax
from jax.experimental import pallas as pl
from jax.experimental.pallas import tpu as pltpu
from jax.experimental.pallas import tpu_sc as plsc

H = 128
EPS = 1e-12
GATHER_W = 128
N_CHUNKS = 4


def _pack_table(word_table):
    u = lax.bitcast_convert_type(word_table, jnp.int32)
    b = lax.shift_right_logical(
        u + 0x7FFF + (lax.shift_right_logical(u, 16) & 1), 16)
    return b[:, : H // 2] | lax.shift_left(b[:, H // 2:], 16)


def _sc_gather(table_pk, ids_2d, n):
    hw = H // 2

    @functools.partial(
        pl.kernel,
        out_type=jax.ShapeDtypeStruct((n // 2, H), jnp.int32),
        mesh=plsc.VectorSubcoreMesh(core_axis_name="core",
                                    subcore_axis_name="subcore"),
        compiler_params=pltpu.CompilerParams(use_tc_tiling_on_sc=False),
    )
    def k(table_hbm, i_hbm, o_hbm):
        def body(i_vmem, o_vmem):
            pltpu.sync_copy(table_hbm.at[i_vmem.at[0]], o_vmem)

        pltpu.emit_pipeline(
            body,
            grid=(n // GATHER_W,),
            in_specs=[pl.BlockSpec((1, GATHER_W), index_map=lambda i: (0, i))],
            out_specs=[pl.BlockSpec((GATHER_W, hw),
                                    index_map=lambda i: (i // 2, i % 2))],
            core_axis_name=("core", "subcore"),
            dimension_semantics=(pltpu.PARALLEL,),
        )(i_hbm, o_hbm)

    return k(table_pk, ids_2d)


def _ln_body(g_ref, pos_ref, tok_ref, gamma_ref, beta_ref, o_ref):
    gi = g_ref[...]
    nseq = o_ref.shape[0]
    seq = o_ref.shape[1]
    hs = seq // 4
    lo = lax.bitcast_convert_type(lax.shift_left(gi, 16), jnp.float32)
    hi = lax.bitcast_convert_type(
        jnp.bitwise_and(gi, jnp.int32(-65536)), jnp.float32)
    parts = []
    for j in range(nseq):
        for g2 in range(2):
            r0 = j * (seq // 2) + g2 * hs
            for h in range(2):
                c = slice(h * (H // 2), (h + 1) * (H // 2))
                parts.append(jnp.concatenate(
                    [lo[r0:r0 + hs, c], hi[r0:r0 + hs, c]], axis=-1))
    x = jnp.concatenate(parts, axis=0)
    pt = pos_ref[...] + tok_ref[...][0][None, :]
    x = (x.reshape(nseq, seq, H) + pt[None]).reshape(nseq * seq, H)
    onesh = jnp.full((H, H), 1.0 / H, dtype=jnp.bfloat16)
    xb = x.astype(jnp.bfloat16)
    mean = jax.lax.dot(xb, onesh, preferred_element_type=jnp.float32)
    ex2 = jax.lax.dot(xb * xb, onesh, preferred_element_type=jnp.float32)
    inv = lax.rsqrt(ex2 - mean * mean + EPS)
    y = (x - mean) * inv
    o_ref[...] = (y * gamma_ref[...][0][None, :]
                  + beta_ref[...][0][None, :]).reshape(nseq, seq, H)


def _ln_body_acc(g_ref, pos_ref, tok_ref, gamma_ref, beta_ref, acc_ref,
                 o_ref):
    del acc_ref
    _ln_body(g_ref, pos_ref, tok_ref, gamma_ref, beta_ref, o_ref)


def _tc_ln_chunk(g, pos, tok, gamma2d, beta2d, out_b, seq, chunk, buf):
    nseq = 16
    rows = g.shape[0]
    rows_blk = nseq * seq // 2
    bc = rows // rows_blk
    base = chunk * bc
    in_specs = [
        pl.BlockSpec((rows_blk, H), lambda i: (i, 0)),
        pl.BlockSpec((seq, H), lambda i: (0, 0)),
        pl.BlockSpec((2, H), lambda i: (0, 0)),
        pl.BlockSpec((1, H), lambda i: (0, 0)),
        pl.BlockSpec((1, H), lambda i: (0, 0)),
    ]
    args = [g, pos, tok, gamma2d, beta2d]
    body = _ln_body
    kwargs = {}
    if buf is not None:
        in_specs.append(pl.BlockSpec(memory_space=pl.ANY))
        args.append(buf)
        body = _ln_body_acc
        kwargs = dict(input_output_aliases={5: 0})
    return pl.pallas_call(
        body,
        grid=(bc,),
        in_specs=in_specs,
        out_specs=pl.BlockSpec((nseq, seq, H), lambda i: (base + i, 0, 0)),
        out_shape=jax.ShapeDtypeStruct((out_b, seq, H), jnp.float32),
        **kwargs,
    )(*args)


def kernel(input_ids, word_table, pos_table, tok_table, gamma, beta):
    B, S = input_ids.shape
    ids = input_ids.astype(jnp.int32)
    gamma2d, beta2d = gamma.reshape(1, H), beta.reshape(1, H)
    wt_pk = _pack_table(word_table)
    bc = B // N_CHUNKS
    gs = []
    for k in range(N_CHUNKS):
        ids_k = ids[k * bc:(k + 1) * bc].reshape(1, bc * S)
        gs.append(_sc_gather(wt_pk, ids_k, bc * S))
    out = None
    for k in range(N_CHUNKS):
        out = _tc_ln_chunk(gs[k], pos_table, tok_table, gamma2d, beta2d,
                           B, S, k, out)
    return out
